# baseline (speedup 1.0000x reference)
import functools

import jax
import jax.numpy as jnp
from jax.experimental import pallas as pl
from jax.experimental.pallas import tpu as pltpu


def _se_kernel(x_ref, w1_ref, w2_ref, o_ref, *, inv_hw):
    xb = x_ref[0]
    pooled = (jnp.sum(xb, axis=(1, 2), keepdims=False)
              * inv_hw)[:, None]
    h = jnp.maximum(
        jax.lax.dot_general(w1_ref[...], pooled, (((1,), (0,)), ((), ())),
                            preferred_element_type=jnp.float32), 0.0)
    s = jax.nn.sigmoid(
        jax.lax.dot_general(w2_ref[...], h, (((1,), (0,)), ((), ())),
                            preferred_element_type=jnp.float32))
    o_ref[0] = xb * s[:, :, None]


def kernel(x, w1, w2):
    B, C, H, W = x.shape
    Cr = w1.shape[0]

    body = functools.partial(_se_kernel, inv_hw=1.0 / float(H * W))
    return pl.pallas_call(
        body,
        out_shape=jax.ShapeDtypeStruct((B, C, H, W), x.dtype),
        grid=(B,),
        in_specs=[
            pl.BlockSpec((1, C, H, W), lambda b: (b, 0, 0, 0)),
            pl.BlockSpec((Cr, C), lambda b: (0, 0)),
            pl.BlockSpec((C, Cr), lambda b: (0, 0)),
        ],
        out_specs=pl.BlockSpec((1, C, H, W), lambda b: (b, 0, 0, 0)),
        compiler_params=pltpu.CompilerParams(
            dimension_semantics=("parallel",),
        ),
    )(x, w1, w2)
